# baseline (speedup 1.0000x reference)
import functools
import jax
import jax.numpy as jnp
from jax import lax
from jax.experimental import pallas as pl
from jax.experimental.pallas import tpu as pltpu
from jax.experimental.pallas import tpu_sc as plsc

_HF = 112
_NW = 32
_C0 = 240
_NSH = 144
_IMGS = 2 * _NSH
_PER_W = _IMGS // _NW
_PAIRS = _PER_W // 2


def _rotate(buf, g):
    @pl.when(g == 1)
    def _():
        def row(r, carry):
            for c in range(7):
                wl = pl.ds(16 * c, 16)
                wr = pl.ds(_HF + 16 * c, 16)
                tmp = buf[r, wl]
                buf[r, wl] = buf[r, wr]
                buf[r, wr] = buf[r + _HF, wl]
                buf[r + _HF, wl] = buf[r + _HF, wr]
                buf[r + _HF, wr] = tmp
            return carry

        lax.fori_loop(0, _HF, row, 0)

    @pl.when(g == 2)
    def _():
        def row(r, carry):
            for c in range(14):
                w = pl.ds(16 * c, 16)
                tmp = buf[r, w]
                buf[r, w] = buf[r + _HF, w]
                buf[r + _HF, w] = tmp
            return carry

        lax.fori_loop(0, _HF, row, 0)

    @pl.when(g == 3)
    def _():
        def row(r, carry):
            for c in range(7):
                wl = pl.ds(16 * c, 16)
                wr = pl.ds(_HF + 16 * c, 16)
                tmp = buf[r, wl]
                buf[r, wl] = buf[r + _HF, wr]
                buf[r + _HF, wr] = buf[r + _HF, wl]
                buf[r + _HF, wl] = buf[r, wr]
                buf[r, wr] = tmp
            return carry

        lax.fori_loop(0, _HF, row, 0)


def kernel(x):
    b, c, h, w = x.shape
    mesh = plsc.VectorSubcoreMesh(core_axis_name="c", subcore_axis_name="s")

    @functools.partial(
        pl.kernel,
        mesh=mesh,
        out_type=jax.ShapeDtypeStruct((b, c, h, w), x.dtype),
        scratch_types=[
            pltpu.VMEM((h, w), x.dtype),
            pltpu.VMEM((h, w), x.dtype),
            pltpu.SemaphoreType.DMA,
            pltpu.SemaphoreType.DMA,
            pltpu.SemaphoreType.DMA,
            pltpu.SemaphoreType.DMA,
        ],
    )
    def k(x_hbm, o_hbm, buf0, buf1, sin0, sin1, sout0, sout1):
        wid = lax.axis_index("s") * 2 + lax.axis_index("c")
        base = wid * _PER_W

        def coords(img):
            ci = _C0 + img % _NSH
            return img // _NSH, ci, ci

        def group(ci):
            return (ci - 192) // 48

        b0, c0, _ = coords(base)
        pltpu.make_async_copy(x_hbm.at[b0, c0], buf0, sin0).start()

        def pair(t, carry):
            i0 = base + 2 * t
            i1 = i0 + 1
            bi0, ci0, co0 = coords(i0)
            bi1, ci1, co1 = coords(i1)

            @pl.when(t > 0)
            def _():
                pltpu.make_async_copy(buf1, o_hbm.at[bi1, co1], sout1).wait()

            pltpu.make_async_copy(x_hbm.at[bi1, ci1], buf1, sin1).start()

            pltpu.make_async_copy(x_hbm.at[bi0, ci0], buf0, sin0).wait()
            _rotate(buf0, group(ci0))
            pltpu.make_async_copy(buf0, o_hbm.at[bi0, co0], sout0).start()

            pltpu.make_async_copy(x_hbm.at[bi1, ci1], buf1, sin1).wait()
            _rotate(buf1, group(ci1))
            pltpu.make_async_copy(buf1, o_hbm.at[bi1, co1], sout1).start()

            @pl.when(t < _PAIRS - 1)
            def _():
                pltpu.make_async_copy(buf0, o_hbm.at[bi0, co0], sout0).wait()
                bn, cn, _ = coords(i0 + 2)
                pltpu.make_async_copy(x_hbm.at[bn, cn], buf0, sin0).start()

            return carry

        lax.fori_loop(0, _PAIRS, pair, 0)

        bl0, cl0, clo0 = coords(base + 2 * _PAIRS - 2)
        pltpu.make_async_copy(buf0, o_hbm.at[bl0, clo0], sout0).wait()
        bt, ct, cto = coords(base + _PER_W - 1)
        pltpu.make_async_copy(x_hbm.at[bt, ct], buf0, sin0).start()
        pltpu.make_async_copy(x_hbm.at[bt, ct], buf0, sin0).wait()
        _rotate(buf0, group(ct))
        pltpu.make_async_copy(buf0, o_hbm.at[bt, cto], sout0).start()

        bl1, cl1, clo1 = coords(base + 2 * _PAIRS - 1)
        pltpu.make_async_copy(buf1, o_hbm.at[bl1, clo1], sout1).wait()
        pltpu.make_async_copy(buf0, o_hbm.at[bt, cto], sout0).wait()

    shuffled = k(x)
    mask = (jnp.arange(c, dtype=jnp.int32) < _C0)[None, :, None, None]
    return jnp.where(mask, x, shuffled)


# speedup vs baseline: 1.1238x; 1.1238x over previous
import functools
import jax
import jax.numpy as jnp
from jax import lax
from jax.experimental import pallas as pl
from jax.experimental.pallas import tpu as pltpu
from jax.experimental.pallas import tpu_sc as plsc

_HF = 112
_NW = 32
_C0 = 240
_NSH = 144
_IMGS = 2 * _NSH
_PER_W = _IMGS // _NW
_PAIRS = _PER_W // 2


def _rotate(buf, g):
    @pl.when(g == 1)
    def _():
        def row(r, carry):
            for c in range(7):
                wl = pl.ds(16 * c, 16)
                wr = pl.ds(_HF + 16 * c, 16)
                tmp = buf[r, wl]
                buf[r, wl] = buf[r, wr]
                buf[r, wr] = buf[r + _HF, wl]
                buf[r + _HF, wl] = buf[r + _HF, wr]
                buf[r + _HF, wr] = tmp
            return carry

        lax.fori_loop(0, _HF, row, 0)

    @pl.when(g == 2)
    def _():
        def row(r, carry):
            for c in range(14):
                w = pl.ds(16 * c, 16)
                tmp = buf[r, w]
                buf[r, w] = buf[r + _HF, w]
                buf[r + _HF, w] = tmp
            return carry

        lax.fori_loop(0, _HF, row, 0)

    @pl.when(g == 3)
    def _():
        def row(r, carry):
            for c in range(7):
                wl = pl.ds(16 * c, 16)
                wr = pl.ds(_HF + 16 * c, 16)
                tmp = buf[r, wl]
                buf[r, wl] = buf[r + _HF, wr]
                buf[r + _HF, wr] = buf[r + _HF, wl]
                buf[r + _HF, wl] = buf[r, wr]
                buf[r, wr] = tmp
            return carry

        lax.fori_loop(0, _HF, row, 0)


def kernel(x):
    b, c, h, w = x.shape
    mesh = plsc.VectorSubcoreMesh(core_axis_name="c", subcore_axis_name="s")

    @functools.partial(
        pl.kernel,
        mesh=mesh,
        out_type=jax.ShapeDtypeStruct((b, c, h, w), x.dtype),
        scratch_types=[
            pltpu.VMEM((h, w), x.dtype),
            pltpu.VMEM((h, w), x.dtype),
            pltpu.SemaphoreType.DMA,
            pltpu.SemaphoreType.DMA,
            pltpu.SemaphoreType.DMA,
            pltpu.SemaphoreType.DMA,
        ],
    )
    def k(x_hbm, o_hbm, buf0, buf1, sin0, sin1, sout0, sout1):
        wid = lax.axis_index("s") * 2 + lax.axis_index("c")
        base = wid * _PER_W

        def coords(img):
            ci = _C0 + img % _NSH
            return img // _NSH, ci, ci

        def group(ci):
            return (ci - 192) // 48

        b0, c0, _ = coords(base)
        pltpu.make_async_copy(x_hbm.at[b0, c0], buf0, sin0).start()

        def pair(t, carry):
            i0 = base + 2 * t
            i1 = i0 + 1
            bi0, ci0, co0 = coords(i0)
            bi1, ci1, co1 = coords(i1)

            @pl.when(t > 0)
            def _():
                pltpu.make_async_copy(buf1, o_hbm.at[bi1, co1], sout1).wait()

            pltpu.make_async_copy(x_hbm.at[bi1, ci1], buf1, sin1).start()

            pltpu.make_async_copy(x_hbm.at[bi0, ci0], buf0, sin0).wait()
            _rotate(buf0, group(ci0))
            pltpu.make_async_copy(buf0, o_hbm.at[bi0, co0], sout0).start()

            pltpu.make_async_copy(x_hbm.at[bi1, ci1], buf1, sin1).wait()
            _rotate(buf1, group(ci1))
            pltpu.make_async_copy(buf1, o_hbm.at[bi1, co1], sout1).start()

            @pl.when(t < _PAIRS - 1)
            def _():
                pltpu.make_async_copy(buf0, o_hbm.at[bi0, co0], sout0).wait()
                bn, cn, _ = coords(i0 + 2)
                pltpu.make_async_copy(x_hbm.at[bn, cn], buf0, sin0).start()

            return carry

        lax.fori_loop(0, _PAIRS, pair, 0)

        bl0, cl0, clo0 = coords(base + 2 * _PAIRS - 2)
        pltpu.make_async_copy(buf0, o_hbm.at[bl0, clo0], sout0).wait()
        bt, ct, cto = coords(base + _PER_W - 1)
        pltpu.make_async_copy(x_hbm.at[bt, ct], buf0, sin0).start()
        pltpu.make_async_copy(x_hbm.at[bt, ct], buf0, sin0).wait()
        _rotate(buf0, group(ct))
        pltpu.make_async_copy(buf0, o_hbm.at[bt, cto], sout0).start()

        bl1, cl1, clo1 = coords(base + 2 * _PAIRS - 1)
        pltpu.make_async_copy(buf1, o_hbm.at[bl1, clo1], sout1).wait()
        pltpu.make_async_copy(buf0, o_hbm.at[bt, cto], sout0).wait()

    shuffled = k(x)

    cblk = 16

    def copy_body(s_ref, x_ref, o_ref):
        del s_ref
        o_ref[...] = x_ref[...]

    blk = pl.BlockSpec((1, cblk, h, w), lambda i, j: (i, j, 0, 0))
    return pl.pallas_call(
        copy_body,
        grid=(b, _C0 // cblk),
        in_specs=[pl.BlockSpec(memory_space=pl.ANY), blk],
        out_specs=blk,
        out_shape=jax.ShapeDtypeStruct(x.shape, x.dtype),
        input_output_aliases={0: 0},
        compiler_params=pltpu.CompilerParams(
            dimension_semantics=("parallel", "parallel"),
        ),
    )(shuffled, x)
